# baseline (speedup 1.0000x reference)
import functools

import jax
import jax.numpy as jnp
from jax import lax
from jax.experimental import pallas as pl
from jax.experimental.pallas import tpu as pltpu
from jax.experimental.pallas import tpu_sc as plsc

PACKET = 64
NPK = 64


def _rank_kernel(x_ref, s_ref, out_ref, *, rows_per_block):
    R = rows_per_block
    x = x_ref[...]
    s = s_ref[...]
    keysT = lax.dot_general(s, x, (((0,), (1,)), ((), ())),
                            precision=lax.Precision.HIGHEST)
    iota_s = lax.broadcasted_iota(jnp.int32, (NPK, R), 0)
    perm_acc = jnp.zeros((NPK, R), jnp.int32)
    for j in range(NPK):
        bj = keysT[j]
        before = (keysT < bj) | ((keysT == bj) & (iota_s < j))
        rank_j = jnp.sum(before.astype(jnp.int32), axis=0)
        perm_acc = perm_acc + jnp.where(iota_s == rank_j, j, 0)
    row = pl.program_id(0) * R + lax.broadcasted_iota(jnp.int32, (NPK, R), 1)
    srcT = row * NPK + perm_acc
    out_ref[...] = srcT.T


def _compute_src_indices(z2d):
    B = z2d.shape[0]
    D = z2d.shape[1]
    R = 128
    sel = (lax.broadcasted_iota(jnp.int32, (D, NPK), 0)
           == lax.broadcasted_iota(jnp.int32, (D, NPK), 1) * PACKET
           ).astype(jnp.float32)
    return pl.pallas_call(
        functools.partial(_rank_kernel, rows_per_block=R),
        grid=(B // R,),
        in_specs=[
            pl.BlockSpec((R, D), lambda i: (i, 0)),
            pl.BlockSpec((D, NPK), lambda i: (0, 0)),
        ],
        out_specs=pl.BlockSpec((R, NPK), lambda i: (i, 0)),
        out_shape=jax.ShapeDtypeStruct((B, NPK), jnp.int32),
    )(z2d, sel)


def _make_sc_gather(n_packets):
    info = plsc.get_sparse_core_info()
    NC, NS = info.num_cores, info.num_subcores
    NW = NC * NS
    per_w = n_packets // NW
    CH = 128
    GRP = 4
    n_groups = per_w // (GRP * CH)
    mesh = plsc.VectorSubcoreMesh(core_axis_name="c", subcore_axis_name="s")

    @functools.partial(
        pl.kernel,
        mesh=mesh,
        out_type=jax.ShapeDtypeStruct((n_packets, PACKET), jnp.float32),
        compiler_params=pltpu.CompilerParams(use_tc_tiling_on_sc=False),
        scratch_types=[
            pltpu.VMEM((per_w,), jnp.int32),
            pltpu.VMEM((2, GRP * CH, PACKET), jnp.float32),
            pltpu.SemaphoreType.DMA,
            pltpu.SemaphoreType.DMA,
            pltpu.SemaphoreType.DMA,
            pltpu.SemaphoreType.DMA,
        ],
    )
    def gather(z2_hbm, idx_hbm, out_hbm, idx_v, buf_v, gsem0, gsem1,
               osem0, osem1):
        wid = lax.axis_index("s") * NC + lax.axis_index("c")
        base = wid * per_w
        pltpu.sync_copy(idx_hbm.at[pl.ds(base, per_w)], idx_v)
        gsems = (gsem0, gsem1)
        osems = (osem0, osem1)

        def fire_gathers(grp, slot):
            cps = []
            for k in range(GRP):
                c = grp * GRP + k
                cp = pltpu.make_async_copy(
                    z2_hbm.at[idx_v.at[pl.ds(c * CH, CH)]],
                    buf_v.at[slot, pl.ds(k * CH, CH)], gsems[slot])
                cp.start()
                cps.append(cp)
            return cps

        def fire_store(grp, slot):
            cp = pltpu.make_async_copy(
                buf_v.at[slot],
                out_hbm.at[pl.ds(base + grp * (GRP * CH), GRP * CH)],
                osems[slot])
            cp.start()
            return cp

        def body(p, _):
            g0 = fire_gathers(2 * p, 0)
            g1 = fire_gathers(2 * p + 1, 1)
            for cp in g0:
                cp.wait()
            s0 = fire_store(2 * p, 0)
            for cp in g1:
                cp.wait()
            s1 = fire_store(2 * p + 1, 1)
            s0.wait()
            s1.wait()
            return 0

        lax.fori_loop(0, n_groups // 2, body, 0)

    return gather


def kernel(z):
    B, _, D = z.shape
    src = _compute_src_indices(z.reshape(B, D))
    z2 = z.reshape(B * NPK, PACKET)
    out2 = _make_sc_gather(B * NPK)(z2, src.reshape(-1))
    return out2.reshape(B, 1, D)
